# baseline (speedup 1.0000x reference)
import functools

import jax
import jax.numpy as jnp
import numpy as np
from jax import lax
from jax.experimental import pallas as pl
from jax.experimental.pallas import tpu as pltpu
from jax.experimental.pallas import tpu_sc as plsc

N_NODES = 10000
N_EDGES = 320000
D_IN = 128
H = 8
D = 16
HD = H * D
AXW = 16

NC = 2
NS = 16
NW = NC * NS
EPW = N_EDGES // NW
CK = 80
NCH = EPW // CK
ECOLS = N_EDGES // CK
CK2 = 20
NCH2 = EPW // CK2
ECOLS2 = N_EDGES // CK2
NPB = 624
REM_BASE = NS * NPB
REM = N_NODES - REM_BASE
ZROWS = 8

_ONES_HD = np.concatenate(
    [np.kron(np.eye(H), np.ones((D, 1))), np.zeros((HD, AXW - H))],
    axis=1).astype(np.float32)
_EXPAND = np.concatenate(
    [np.kron(np.eye(H), np.ones((1, D))), np.zeros((AXW - H, HD))],
    axis=0).astype(np.float32)

_BN = 1000
_BE = 1000


def _proj_body(x_ref, wq_ref, wk_ref, wv_ref, q_ref, k_ref, v_ref):
    xb = x_ref[...]
    q_ref[...] = jnp.dot(xb, wq_ref[...], preferred_element_type=jnp.float32)
    k_ref[...] = jnp.dot(xb, wk_ref[...], preferred_element_type=jnp.float32)
    v_ref[...] = jnp.dot(xb, wv_ref[...], preferred_element_type=jnp.float32)


def _project_qkv(x, WQ, WK, WV):
    bs_w = pl.BlockSpec((D_IN, HD), lambda i: (0, 0))
    return pl.pallas_call(
        _proj_body,
        grid=(N_NODES // _BN,),
        in_specs=[pl.BlockSpec((_BN, D_IN), lambda i: (i, 0)), bs_w, bs_w, bs_w],
        out_specs=[pl.BlockSpec((_BN, HD), lambda i: (i, 0))] * 3,
        out_shape=[jax.ShapeDtypeStruct((N_NODES, HD), jnp.float32)] * 3,
    )(x, WQ, WK, WV)


def _edge_body(ea_ref, g_ref, we_ref, ones_ref, eout_ref, ax_ref):
    t = jnp.dot(ea_ref[...], we_ref[...], preferred_element_type=jnp.float32)
    eo = g_ref[...] * t
    eout_ref[...] = eo
    s = jnp.dot(eo, ones_ref[...], preferred_element_type=jnp.float32)
    ax_ref[...] = jnp.exp(jnp.clip(s, -5.0, 5.0))


def _edge_stage(edge_attr, g, WE):
    return pl.pallas_call(
        _edge_body,
        grid=(N_EDGES // _BE,),
        in_specs=[
            pl.BlockSpec((_BE, D_IN), lambda i: (i, 0)),
            pl.BlockSpec((_BE, HD), lambda i: (i, 0)),
            pl.BlockSpec((D_IN, HD), lambda i: (0, 0)),
            pl.BlockSpec((HD, AXW), lambda i: (0, 0)),
        ],
        out_specs=[
            pl.BlockSpec((_BE, HD), lambda i: (i, 0)),
            pl.BlockSpec((_BE, AXW), lambda i: (i, 0)),
        ],
        out_shape=[
            jax.ShapeDtypeStruct((N_EDGES, HD), jnp.float32),
            jax.ShapeDtypeStruct((N_EDGES, AXW), jnp.float32),
        ],
    )(edge_attr, g, WE, _ONES_HD)


def _final_body(a_ref, b_ref, za_ref, zb_ref, exp_ref, h_ref):
    s = a_ref[...] + b_ref[...]
    z = za_ref[...] + zb_ref[...]
    zfull = jnp.dot(z, exp_ref[...], preferred_element_type=jnp.float32)
    h_ref[...] = s / (zfull + 1e-6)


def _finalize(pv, pz):
    nb = N_NODES // _BN
    return pl.pallas_call(
        _final_body,
        grid=(nb,),
        in_specs=[
            pl.BlockSpec((_BN, HD), lambda i: (i, 0)),
            pl.BlockSpec((_BN, HD), lambda i: (i + nb, 0)),
            pl.BlockSpec((_BN, AXW), lambda i: (i, 0)),
            pl.BlockSpec((_BN, AXW), lambda i: (i + nb, 0)),
            pl.BlockSpec((AXW, HD), lambda i: (0, 0)),
        ],
        out_specs=pl.BlockSpec((_BN, HD), lambda i: (i, 0)),
        out_shape=jax.ShapeDtypeStruct((N_NODES, HD), jnp.float32),
    )(pv, pv, pz, pz, _EXPAND)


def _sc_gather_alpha_body(k_hbm, q_hbm, row_hbm, col_hbm, g_hbm,
                          rix, cix,
                          krows0, qrows0, gout0,
                          krows1, qrows1, gout1,
                          semg0, semg1, semw0, semw1):
    wid = lax.axis_index("s") * NC + lax.axis_index("c")
    base0 = wid * EPW
    crow0 = wid * NCH

    pltpu.sync_copy(row_hbm.at[pl.ds(crow0, NCH)], rix)
    pltpu.sync_copy(col_hbm.at[pl.ds(crow0, NCH)], cix)

    def fire(j, krows, qrows, sem):
        pltpu.async_copy(k_hbm.at[rix.at[j]], krows, sem)
        pltpu.async_copy(q_hbm.at[cix.at[j]], qrows, sem)

    def drain_gather(j, krows, qrows, sem):
        pltpu.make_async_copy(k_hbm.at[rix.at[j]], krows, sem).wait()
        pltpu.make_async_copy(q_hbm.at[cix.at[j]], qrows, sem).wait()

    def compute(krows, qrows, gout):
        def row_body(e, carry):
            for h in range(H):
                sl = pl.ds(h * D, D)
                gout[e, sl] = jnp.clip(krows[e, sl] * qrows[e, sl] * 0.25,
                                       -5.0, 5.0)
            return carry

        lax.fori_loop(0, CK, row_body, 0, unroll=2)

    def store(j, gout, sem):
        pltpu.async_copy(gout, g_hbm.at[pl.ds(base0 + j * CK, CK)], sem)

    def drain_store(j, gout, sem):
        pltpu.make_async_copy(gout, g_hbm.at[pl.ds(base0 + j * CK, CK)],
                              sem).wait()

    fire(0, krows0, qrows0, semg0)

    def outer(i2, carry):
        j0 = 2 * i2
        j1 = j0 + 1
        fire(j1, krows1, qrows1, semg1)
        drain_gather(j0, krows0, qrows0, semg0)

        @pl.when(i2 > 0)
        def _():
            drain_store(j0 - 2, gout0, semw0)

        compute(krows0, qrows0, gout0)
        store(j0, gout0, semw0)

        @pl.when(i2 < NCH // 2 - 1)
        def _():
            fire(j0 + 2, krows0, qrows0, semg0)

        drain_gather(j1, krows1, qrows1, semg1)

        @pl.when(i2 > 0)
        def _():
            drain_store(j1 - 2, gout1, semw1)

        compute(krows1, qrows1, gout1)
        store(j1, gout1, semw1)
        return carry

    lax.fori_loop(0, NCH // 2, outer, 0)

    jl = NCH - 1
    fire(jl, krows0, qrows0, semg0)
    drain_gather(jl, krows0, qrows0, semg0)
    drain_store(jl - 3, gout0, semw0)
    compute(krows0, qrows0, gout0)
    store(jl, gout0, semw0)
    drain_store(jl, gout0, semw0)
    drain_store(jl - 2, gout1, semw1)


def _sc_aggregate_body(v_hbm, ax_hbm, row_hbm, col_hbm, pv_hbm, pz_hbm,
                       rix, cix,
                       vrows0, axr0, cv0, azr0,
                       vrows1, axr1, cv1, azr1,
                       zv, zz, accv, accz,
                       semg0, semg1, sems0, sems1):
    cid = lax.axis_index("c")
    sid = lax.axis_index("s")
    wid = sid * NC + cid
    base0 = wid * EPW
    crow0 = wid * NCH2
    nbase = sid * NPB

    def zrow_v(r, carry):
        for j in range(HD // D):
            zv[r, pl.ds(j * D, D)] = jnp.zeros((D,), jnp.float32)
        zz[r, :] = jnp.zeros((AXW,), jnp.float32)
        return carry

    lax.fori_loop(0, ZROWS, zrow_v, 0)

    def zcopy(t, carry):
        pltpu.sync_copy(zv, accv.at[pl.ds(nbase + t * ZROWS, ZROWS)])
        pltpu.sync_copy(zz, accz.at[pl.ds(nbase + t * ZROWS, ZROWS)])
        return carry

    lax.fori_loop(0, NPB // ZROWS, zcopy, 0)

    @pl.when(sid == NS - 1)
    def _zero_rem():
        for t in range(REM // ZROWS):
            pltpu.sync_copy(zv, accv.at[pl.ds(REM_BASE + t * ZROWS, ZROWS)])
            pltpu.sync_copy(zz, accz.at[pl.ds(REM_BASE + t * ZROWS, ZROWS)])

    pltpu.sync_copy(row_hbm.at[pl.ds(crow0, NCH2)], rix)
    pltpu.sync_copy(col_hbm.at[pl.ds(crow0, NCH2)], cix)
    plsc.subcore_barrier()

    def fire(j, vrows, axr, sem):
        pltpu.async_copy(v_hbm.at[rix.at[j]], vrows, sem)
        pltpu.async_copy(ax_hbm.at[pl.ds(base0 + j * CK2, CK2)], axr, sem)

    def drain_gather(j, vrows, axr, sem):
        pltpu.make_async_copy(v_hbm.at[rix.at[j]], vrows, sem).wait()
        pltpu.make_async_copy(ax_hbm.at[pl.ds(base0 + j * CK2, CK2)], axr,
                              sem).wait()

    def compute(vrows, axr, cv, azr):
        def row_body(e, carry):
            ax16 = axr[e, :]
            azr[e, :] = ax16
            for h in range(H):
                sl = pl.ds(h * D, D)
                cv[e, sl] = vrows[e, sl] * ax16[h]
            return carry

        lax.fori_loop(0, CK2, row_body, 0, unroll=2)

    def scatter(j, cv, azr, sem):
        pltpu.async_copy(cv, accv.at[cix.at[j]], sem, add=True)
        pltpu.async_copy(azr, accz.at[cix.at[j]], sem, add=True)

    def drain_scatter(j, cv, azr, sem):
        pltpu.make_async_copy(cv, accv.at[cix.at[j]], sem).wait()
        pltpu.make_async_copy(azr, accz.at[cix.at[j]], sem).wait()

    fire(0, vrows0, axr0, semg0)

    def outer(i2, carry):
        j0 = 2 * i2
        j1 = j0 + 1
        fire(j1, vrows1, axr1, semg1)
        drain_gather(j0, vrows0, axr0, semg0)

        @pl.when(i2 > 0)
        def _():
            drain_scatter(j0 - 2, cv0, azr0, sems0)

        compute(vrows0, axr0, cv0, azr0)
        scatter(j0, cv0, azr0, sems0)

        @pl.when(i2 < NCH2 // 2 - 1)
        def _():
            fire(j0 + 2, vrows0, axr0, semg0)

        drain_gather(j1, vrows1, axr1, semg1)

        @pl.when(i2 > 0)
        def _():
            drain_scatter(j1 - 2, cv1, azr1, sems1)

        compute(vrows1, axr1, cv1, azr1)
        scatter(j1, cv1, azr1, sems1)
        return carry

    lax.fori_loop(0, NCH2 // 2, outer, 0)

    drain_scatter(NCH2 - 2, cv0, azr0, sems0)
    drain_scatter(NCH2 - 1, cv1, azr1, sems1)

    plsc.subcore_barrier()
    pltpu.sync_copy(accv.at[pl.ds(nbase, NPB)],
                    pv_hbm.at[pl.ds(cid * N_NODES + nbase, NPB)])
    pltpu.sync_copy(accz.at[pl.ds(nbase, NPB)],
                    pz_hbm.at[pl.ds(cid * N_NODES + nbase, NPB)])

    @pl.when(sid == NS - 1)
    def _dump_rem():
        pltpu.sync_copy(accv.at[pl.ds(REM_BASE, REM)],
                        pv_hbm.at[pl.ds(cid * N_NODES + REM_BASE, REM)])
        pltpu.sync_copy(accz.at[pl.ds(REM_BASE, REM)],
                        pz_hbm.at[pl.ds(cid * N_NODES + REM_BASE, REM)])


@functools.cache
def _sc_kernels():
    mesh = plsc.VectorSubcoreMesh(core_axis_name="c", subcore_axis_name="s",
                                  num_cores=NC, num_subcores=NS)
    scp = pltpu.CompilerParams(use_tc_tiling_on_sc=False)
    gather_alpha = pl.kernel(
        _sc_gather_alpha_body,
        mesh=mesh,
        compiler_params=scp,
        out_type=jax.ShapeDtypeStruct((N_EDGES, HD), jnp.float32),
        scratch_types=[
            pltpu.VMEM((NCH, CK), jnp.int32),
            pltpu.VMEM((NCH, CK), jnp.int32),
            pltpu.VMEM((CK, HD), jnp.float32),
            pltpu.VMEM((CK, HD), jnp.float32),
            pltpu.VMEM((CK, HD), jnp.float32),
            pltpu.VMEM((CK, HD), jnp.float32),
            pltpu.VMEM((CK, HD), jnp.float32),
            pltpu.VMEM((CK, HD), jnp.float32),
            pltpu.SemaphoreType.DMA,
            pltpu.SemaphoreType.DMA,
            pltpu.SemaphoreType.DMA,
            pltpu.SemaphoreType.DMA,
        ],
    )
    aggregate = pl.kernel(
        _sc_aggregate_body,
        mesh=mesh,
        compiler_params=scp,
        out_type=(
            jax.ShapeDtypeStruct((NC * N_NODES, HD), jnp.float32),
            jax.ShapeDtypeStruct((NC * N_NODES, AXW), jnp.float32),
        ),
        scratch_types=[
            pltpu.VMEM((NCH2, CK2), jnp.int32),
            pltpu.VMEM((NCH2, CK2), jnp.int32),
            pltpu.VMEM((CK2, HD), jnp.float32),
            pltpu.VMEM((CK2, AXW), jnp.float32),
            pltpu.VMEM((CK2, HD), jnp.float32),
            pltpu.VMEM((CK2, AXW), jnp.float32),
            pltpu.VMEM((CK2, HD), jnp.float32),
            pltpu.VMEM((CK2, AXW), jnp.float32),
            pltpu.VMEM((CK2, HD), jnp.float32),
            pltpu.VMEM((CK2, AXW), jnp.float32),
            pltpu.VMEM((ZROWS, HD), jnp.float32),
            pltpu.VMEM((ZROWS, AXW), jnp.float32),
            pltpu.VMEM_SHARED((N_NODES, HD), jnp.float32),
            pltpu.VMEM_SHARED((N_NODES, AXW), jnp.float32),
            pltpu.SemaphoreType.DMA,
            pltpu.SemaphoreType.DMA,
            pltpu.SemaphoreType.DMA,
            pltpu.SemaphoreType.DMA,
        ],
    )
    return gather_alpha, aggregate


def kernel(x, edge_attr, edge_index, WQ, WK, WV, WE):
    gather_alpha, aggregate = _sc_kernels()
    row2d = edge_index[0].reshape(ECOLS, CK)
    col2d = edge_index[1].reshape(ECOLS, CK)
    row2b = edge_index[0].reshape(ECOLS2, CK2)
    col2b = edge_index[1].reshape(ECOLS2, CK2)
    q, k, v = _project_qkv(x, WQ, WK, WV)
    g = gather_alpha(k, q, row2d, col2d)
    e_out, ax = _edge_stage(edge_attr, g, WE)
    pv, pz = aggregate(v, ax, row2b, col2b)
    h = _finalize(pv, pz)
    return (h.reshape(N_NODES, H, D), e_out.reshape(N_EDGES, H, D))
